# baseline (speedup 1.0000x reference)
import functools

import jax
import jax.numpy as jnp
from jax import lax
from jax.experimental import pallas as pl
from jax.experimental.pallas import tpu as pltpu
from jax.experimental.pallas import tpu_sc as plsc

B, T, C = 4, 2048, 1024


@functools.partial(jax.jit, static_argnames=())
def _positional_broadcast(table):
    info = plsc.get_sparse_core_info()
    nw = info.num_cores * info.num_subcores
    rows = T // nw

    mesh = plsc.VectorSubcoreMesh(core_axis_name="c", subcore_axis_name="s")

    @functools.partial(
        pl.kernel,
        mesh=mesh,
        out_type=jax.ShapeDtypeStruct((B, T, C), jnp.float32),
        scratch_types=[
            pltpu.SemaphoreType.DMA,
        ],
    )
    def body(table_hbm, out_hbm, sem):
        wid = lax.axis_index("s") * info.num_cores + lax.axis_index("c")
        base = wid * rows
        copies = [
            pltpu.async_copy(
                table_hbm.at[pl.ds(base, rows)],
                out_hbm.at[b, pl.ds(base, rows)],
                sem,
            )
            for b in range(B)
        ]
        for c in copies:
            c.wait()

    return body(table)


def kernel(x, table):
    del x
    return _positional_broadcast(table)


# speedup vs baseline: 30.9836x; 30.9836x over previous
import functools

import jax
import jax.numpy as jnp
from jax import lax
from jax.experimental import pallas as pl
from jax.experimental.pallas import tpu as pltpu
from jax.experimental.pallas import tpu_sc as plsc

B, T, C = 4, 2048, 1024


@functools.partial(jax.jit, static_argnames=())
def _positional_broadcast(table):
    info = plsc.get_sparse_core_info()
    nw = info.num_cores * info.num_subcores
    rows = T // nw

    mesh = plsc.VectorSubcoreMesh(core_axis_name="c", subcore_axis_name="s")

    @functools.partial(
        pl.kernel,
        mesh=mesh,
        out_type=jax.ShapeDtypeStruct((B, T, C), jnp.float32),
        scratch_types=[
            pltpu.VMEM((2, rows // 4, C), jnp.float32),
            pltpu.SemaphoreType.DMA,
            pltpu.SemaphoreType.DMA,
        ],
    )
    def body(table_hbm, out_hbm, buf, rsem, wsem):
        wid = lax.axis_index("s") * info.num_cores + lax.axis_index("c")
        nch = 4
        ch = rows // nch
        base = wid * rows

        def read(i):
            return pltpu.async_copy(
                table_hbm.at[pl.ds(base + i * ch, ch)], buf.at[i % 2], rsem
            )

        def writes(i):
            return [
                pltpu.async_copy(
                    buf.at[i % 2], out_hbm.at[b, pl.ds(base + i * ch, ch)], wsem
                )
                for b in range(B)
            ]

        pending_w = []
        rd = read(0)
        for i in range(nch):
            rd.wait()
            if i >= 1:
                for c in pending_w[i - 1]:
                    c.wait()
            if i + 1 < nch:
                rd = read(i + 1)
            pending_w.append(writes(i))
        for c in pending_w[nch - 1]:
            c.wait()

    return body(table)


def kernel(x, table):
    del x
    return _positional_broadcast(table)
